# Initial kernel scaffold; baseline (speedup 1.0000x reference)
import jax
import jax.numpy as jnp
from jax.experimental import pallas as pl


def kernel(x, edge_index, edge_attr, batch, W_emb, b_emb, conv_params, W_fc, b_fc, W_out, b_out):
    raise NotImplementedError("write your pallas kernel here")


# speedup vs baseline: 1.5540x; 1.5540x over previous
import functools

import jax
import jax.numpy as jnp
from jax import lax
from jax.experimental import pallas as pl
from jax.experimental.pallas import tpu as pltpu
from jax.experimental.pallas import tpu_sc as plsc

F32 = jnp.float32


def _embed_body(x_ref, w_ref, b_ref, o_ref):
    o_ref[...] = (
        jnp.dot(x_ref[...], w_ref[...], preferred_element_type=F32) + b_ref[...]
    )


def _embed(x, W, b):
    return pl.pallas_call(
        _embed_body,
        out_shape=jax.ShapeDtypeStruct((x.shape[0], W.shape[1]), F32),
    )(x, W, b.reshape(1, -1))


def _tables_body(h_ref, wd_ref, ws_ref, oa_ref, ob_ref):
    h = h_ref[...]
    oa_ref[...] = jnp.dot(h, wd_ref[...], preferred_element_type=F32)
    ob_ref[...] = jnp.dot(h, ws_ref[...], preferred_element_type=F32)


def _tables(h, Wd, Wsr):
    n = h.shape[0]
    return pl.pallas_call(
        _tables_body,
        out_shape=(
            jax.ShapeDtypeStruct((n, Wd.shape[1]), F32),
            jax.ShapeDtypeStruct((n, Wsr.shape[1]), F32),
        ),
    )(h, Wd, Wsr)


def _ee_body(ea_ref, w_ref, b_ref, o_ref):
    o_ref[...] = (
        jnp.dot(ea_ref[...], w_ref[...], preferred_element_type=F32) + b_ref[...]
    )


def _edge_tables(ea, We, bcat):
    e, de = ea.shape
    rows = 8000
    return pl.pallas_call(
        _ee_body,
        grid=(e // rows,),
        in_specs=[
            pl.BlockSpec((rows, de), lambda i: (i, 0)),
            pl.BlockSpec((de, We.shape[1]), lambda i: (0, 0)),
            pl.BlockSpec((1, We.shape[1]), lambda i: (0, 0)),
        ],
        out_specs=pl.BlockSpec((rows, We.shape[1]), lambda i: (i, 0)),
        out_shape=jax.ShapeDtypeStruct((e, We.shape[1]), F32),
    )(ea, We, bcat.reshape(1, -1))


def _bn_body(h_ref, agg_ref, g_ref, be_ref, o_ref):
    agg = agg_ref[...]
    m = jnp.mean(agg, axis=0, keepdims=True)
    c = agg - m
    v = jnp.mean(c * c, axis=0, keepdims=True)
    o_ref[...] = h_ref[...] + c * lax.rsqrt(v + 1e-5) * g_ref[...] + be_ref[...]


def _bn(h, agg, gamma, beta):
    return pl.pallas_call(
        _bn_body,
        out_shape=jax.ShapeDtypeStruct(h.shape, F32),
    )(h, agg, gamma.reshape(1, -1), beta.reshape(1, -1))


def _head_body(h_ref, b_ref, wfc_ref, bfc_ref, wout_ref, bout_ref, o_ref):
    h = h_ref[...]
    n = h.shape[0]
    g = o_ref.shape[0]
    bids = b_ref[...]
    gids = lax.broadcasted_iota(jnp.int32, (n, g), 1)
    onehot = (bids == gids).astype(F32)
    sums = lax.dot_general(
        onehot, h, (((0,), (0,)), ((), ())), preferred_element_type=F32
    )
    counts = jnp.sum(onehot, axis=0)[:, None]
    crys = sums / jnp.maximum(counts, 1.0)
    crys = jax.nn.softplus(crys)
    crys = jnp.dot(crys, wfc_ref[...], preferred_element_type=F32) + bfc_ref[...]
    crys = jax.nn.softplus(crys)
    o_ref[...] = (
        jnp.dot(crys, wout_ref[...], preferred_element_type=F32) + bout_ref[...]
    )


def _head(h, batch, G, W_fc, b_fc, W_out, b_out):
    return pl.pallas_call(
        _head_body,
        out_shape=jax.ShapeDtypeStruct((G, W_out.shape[1]), F32),
    )(
        h,
        batch.reshape(-1, 1),
        W_fc,
        b_fc.reshape(1, -1),
        W_out,
        b_out.reshape(1, -1),
    )


def _softplus16(t):
    u = jnp.exp(-jnp.abs(t))
    v = u / (2.0 + u)
    v2 = v * v
    p = jnp.full_like(t, 2.0 / 9.0)
    for c in (2.0 / 7.0, 2.0 / 5.0, 2.0 / 3.0, 2.0):
        p = p * v2 + c
    return jnp.maximum(t, 0.0) + p * v


def _make_sc_edge(n, e, dh, c):
    ns = 16
    nw = 32
    ept = e // nw
    steps = ept // c
    assert ept * nw == e and steps * c == ept and c % 8 == 0
    d2 = 2 * dh

    mesh = plsc.VectorSubcoreMesh(core_axis_name="c", subcore_axis_name="s")

    @functools.partial(
        pl.kernel,
        out_type=jax.ShapeDtypeStruct((e, dh), F32),
        mesh=mesh,
        scratch_types=[
            pltpu.VMEM((c,), jnp.int32),
            pltpu.VMEM((c,), jnp.int32),
            pltpu.VMEM((c, d2), F32),
            pltpu.VMEM((c, d2), F32),
            pltpu.VMEM((c, d2), F32),
            pltpu.VMEM((c, dh), F32),
            pltpu.SemaphoreType.DMA,
        ],
    )
    def sc_edge(
        a_hbm,
        b_hbm,
        ee_hbm,
        dst_hbm,
        src_hbm,
        out_hbm,
        dsti,
        srci,
        eeb,
        ab,
        bb,
        msgb,
        sem,
    ):
        cid = lax.axis_index("c")
        sid = lax.axis_index("s")
        wid = cid * ns + sid

        @pl.loop(0, steps)
        def _(g):
            base = wid * ept + g * c
            pltpu.sync_copy(dst_hbm.at[pl.ds(base, c)], dsti)
            pltpu.sync_copy(src_hbm.at[pl.ds(base, c)], srci)
            pltpu.sync_copy(ee_hbm.at[pl.ds(base, c)], eeb)
            pltpu.async_copy(a_hbm.at[dsti], ab, sem).wait()
            pltpu.async_copy(b_hbm.at[srci], bb, sem).wait()

            @pl.loop(0, c)
            def _(r):
                for k in range(dh // 16):
                    sf = pl.ds(k * 16, 16)
                    ss = pl.ds(dh + k * 16, 16)
                    tf = eeb[r, sf] + ab[r, sf] + bb[r, sf]
                    ts = eeb[r, ss] + ab[r, ss] + bb[r, ss]
                    sig = 1.0 / (1.0 + jnp.exp(-tf))
                    msgb[r, sf] = sig * _softplus16(ts)

            pltpu.sync_copy(msgb, out_hbm.at[pl.ds(base, c)])

    return sc_edge


def kernel(
    x,
    edge_index,
    edge_attr,
    batch,
    W_emb,
    b_emb,
    conv_params,
    W_fc,
    b_fc,
    W_out,
    b_out,
):
    n = x.shape[0]
    e = edge_attr.shape[0]
    dh = W_emb.shape[1]
    src = edge_index[0]
    dst = edge_index[1]

    h = _embed(x, W_emb, b_emb)
    sc_edge = _make_sc_edge(n, e, dh, 80)

    for Wf, bf, Ws, bs, gamma, beta in conv_params:
        Wd = jnp.concatenate([Wf[:dh], Ws[:dh]], axis=1)
        Wsr = jnp.concatenate([Wf[dh : 2 * dh], Ws[dh : 2 * dh]], axis=1)
        We = jnp.concatenate([Wf[2 * dh :], Ws[2 * dh :]], axis=1)
        bcat = jnp.concatenate([bf, bs])
        A, B = _tables(h, Wd, Wsr)
        EE = _edge_tables(edge_attr, We, bcat)
        msg = sc_edge(A, B, EE, dst, src)
        agg = jnp.zeros((n, dh), F32).at[dst].add(msg)
        h = _bn(h, agg, gamma, beta)

    return _head(h, batch, 128, W_fc, b_fc, W_out, b_out)


# speedup vs baseline: 1.9184x; 1.2345x over previous
import functools

import jax
import jax.numpy as jnp
from jax import lax
from jax.experimental import pallas as pl
from jax.experimental.pallas import tpu as pltpu
from jax.experimental.pallas import tpu_sc as plsc

F32 = jnp.float32


def _embed_body(x_ref, w_ref, b_ref, o_ref):
    o_ref[...] = (
        jnp.dot(x_ref[...], w_ref[...], preferred_element_type=F32) + b_ref[...]
    )


def _embed(x, W, b):
    return pl.pallas_call(
        _embed_body,
        out_shape=jax.ShapeDtypeStruct((x.shape[0], W.shape[1]), F32),
    )(x, W, b.reshape(1, -1))


def _tables_body(h_ref, wd_ref, ws_ref, oa_ref, ob_ref):
    h = h_ref[...]
    oa_ref[...] = jnp.dot(h, wd_ref[...], preferred_element_type=F32)
    ob_ref[...] = jnp.dot(h, ws_ref[...], preferred_element_type=F32)


def _tables(h, Wd, Wsr):
    n = h.shape[0]
    return pl.pallas_call(
        _tables_body,
        out_shape=(
            jax.ShapeDtypeStruct((n, Wd.shape[1]), F32),
            jax.ShapeDtypeStruct((n, Wsr.shape[1]), F32),
        ),
    )(h, Wd, Wsr)


def _ee_body(ea_ref, w_ref, b_ref, o_ref):
    o_ref[...] = (
        jnp.dot(ea_ref[...], w_ref[...], preferred_element_type=F32) + b_ref[...]
    )


def _edge_tables(ea, We, bcat):
    e, de = ea.shape
    rows = 8000
    return pl.pallas_call(
        _ee_body,
        grid=(e // rows,),
        in_specs=[
            pl.BlockSpec((rows, de), lambda i: (i, 0)),
            pl.BlockSpec((de, We.shape[1]), lambda i: (0, 0)),
            pl.BlockSpec((1, We.shape[1]), lambda i: (0, 0)),
        ],
        out_specs=pl.BlockSpec((rows, We.shape[1]), lambda i: (i, 0)),
        out_shape=jax.ShapeDtypeStruct((e, We.shape[1]), F32),
    )(ea, We, bcat.reshape(1, -1))


def _bn_body(h_ref, agg_ref, g_ref, be_ref, o_ref):
    agg = agg_ref[...]
    m = jnp.mean(agg, axis=0, keepdims=True)
    c = agg - m
    v = jnp.mean(c * c, axis=0, keepdims=True)
    o_ref[...] = h_ref[...] + c * lax.rsqrt(v + 1e-5) * g_ref[...] + be_ref[...]


def _bn(h, agg, gamma, beta):
    return pl.pallas_call(
        _bn_body,
        out_shape=jax.ShapeDtypeStruct(h.shape, F32),
    )(h, agg, gamma.reshape(1, -1), beta.reshape(1, -1))


def _head_body(h_ref, b_ref, wfc_ref, bfc_ref, wout_ref, bout_ref, o_ref):
    h = h_ref[...]
    n = h.shape[0]
    g = o_ref.shape[0]
    bids = b_ref[...]
    gids = lax.broadcasted_iota(jnp.int32, (n, g), 1)
    onehot = (bids == gids).astype(F32)
    sums = lax.dot_general(
        onehot, h, (((0,), (0,)), ((), ())), preferred_element_type=F32
    )
    counts = jnp.sum(onehot, axis=0)[:, None]
    crys = sums / jnp.maximum(counts, 1.0)
    crys = jax.nn.softplus(crys)
    crys = jnp.dot(crys, wfc_ref[...], preferred_element_type=F32) + bfc_ref[...]
    crys = jax.nn.softplus(crys)
    o_ref[...] = (
        jnp.dot(crys, wout_ref[...], preferred_element_type=F32) + bout_ref[...]
    )


def _head(h, batch, G, W_fc, b_fc, W_out, b_out):
    return pl.pallas_call(
        _head_body,
        out_shape=jax.ShapeDtypeStruct((G, W_out.shape[1]), F32),
    )(
        h,
        batch.reshape(-1, 1),
        W_fc,
        b_fc.reshape(1, -1),
        W_out,
        b_out.reshape(1, -1),
    )


def _softplus16(t):
    u = jnp.exp(-jnp.abs(t))
    v = u / (2.0 + u)
    v2 = v * v
    p = jnp.full_like(t, 2.0 / 9.0)
    for c in (2.0 / 7.0, 2.0 / 5.0, 2.0 / 3.0, 2.0):
        p = p * v2 + c
    return jnp.maximum(t, 0.0) + p * v


def _make_sc_edge(n, e, dh, c):
    ns = 16
    nw = 32
    ept = e // nw
    steps = ept // c
    assert ept * nw == e and steps * c == ept and c % 8 == 0
    d2 = 2 * dh

    mesh = plsc.VectorSubcoreMesh(core_axis_name="c", subcore_axis_name="s")

    @functools.partial(
        pl.kernel,
        out_type=jax.ShapeDtypeStruct((e, dh), F32),
        mesh=mesh,
        scratch_types=[
            pltpu.VMEM((ept,), jnp.int32),
            pltpu.VMEM((ept,), jnp.int32),
            pltpu.VMEM((c, d2), F32),
            pltpu.VMEM((c, d2), F32),
            pltpu.VMEM((c, d2), F32),
            pltpu.VMEM((c, dh), F32),
            pltpu.SemaphoreType.DMA,
            pltpu.SemaphoreType.DMA,
            pltpu.SemaphoreType.DMA,
        ],
    )
    def sc_edge(
        a_hbm,
        b_hbm,
        ee_hbm,
        dst_hbm,
        src_hbm,
        out_hbm,
        dsti,
        srci,
        eeb,
        ab,
        bb,
        msgb,
        sem_a,
        sem_b,
        sem_e,
    ):
        cid = lax.axis_index("c")
        sid = lax.axis_index("s")
        wid = cid * ns + sid

        pltpu.sync_copy(dst_hbm.at[pl.ds(wid * ept, ept)], dsti)
        pltpu.sync_copy(src_hbm.at[pl.ds(wid * ept, ept)], srci)

        @pl.loop(0, steps)
        def _(g):
            base = wid * ept + g * c
            cp_e = pltpu.async_copy(ee_hbm.at[pl.ds(base, c)], eeb, sem_e)
            cp_a = pltpu.async_copy(a_hbm.at[dsti.at[pl.ds(g * c, c)]], ab, sem_a)
            cp_b = pltpu.async_copy(b_hbm.at[srci.at[pl.ds(g * c, c)]], bb, sem_b)
            cp_e.wait()
            cp_a.wait()
            cp_b.wait()

            @pl.loop(0, c)
            def _(r):
                for k in range(dh // 16):
                    sf = pl.ds(k * 16, 16)
                    ss = pl.ds(dh + k * 16, 16)
                    tf = eeb[r, sf] + ab[r, sf] + bb[r, sf]
                    ts = eeb[r, ss] + ab[r, ss] + bb[r, ss]
                    sig = 1.0 / (1.0 + jnp.exp(-tf))
                    msgb[r, sf] = sig * _softplus16(ts)

            pltpu.sync_copy(msgb, out_hbm.at[pl.ds(base, c)])

    return sc_edge


def kernel(
    x,
    edge_index,
    edge_attr,
    batch,
    W_emb,
    b_emb,
    conv_params,
    W_fc,
    b_fc,
    W_out,
    b_out,
):
    n = x.shape[0]
    e = edge_attr.shape[0]
    dh = W_emb.shape[1]
    src = edge_index[0]
    dst = edge_index[1]

    h = _embed(x, W_emb, b_emb)
    sc_edge = _make_sc_edge(n, e, dh, 80)

    for Wf, bf, Ws, bs, gamma, beta in conv_params:
        Wd = jnp.concatenate([Wf[:dh], Ws[:dh]], axis=1)
        Wsr = jnp.concatenate([Wf[dh : 2 * dh], Ws[dh : 2 * dh]], axis=1)
        We = jnp.concatenate([Wf[2 * dh :], Ws[2 * dh :]], axis=1)
        bcat = jnp.concatenate([bf, bs])
        A, B = _tables(h, Wd, Wsr)
        EE = _edge_tables(edge_attr, We, bcat)
        msg = sc_edge(A, B, EE, dst, src)
        agg = jnp.zeros((n, dh), F32).at[dst].add(msg)
        h = _bn(h, agg, gamma, beta)

    return _head(h, batch, 128, W_fc, b_fc, W_out, b_out)
